# Initial kernel scaffold; baseline (speedup 1.0000x reference)
import jax
import jax.numpy as jnp
from jax.experimental import pallas as pl


def kernel(u, time_delta, time_bar, time_cur, significance, magnitudo, z, neigh, w_t, alpha, psi, omega_w, omega_b, W_h_w, W_h_b, W_e2n_w, W_e2n_b, W_rec_e_w, W_rec_e_b, W_rec_n_w, W_rec_n_b, W_t_w, W_t_b):
    raise NotImplementedError("write your pallas kernel here")


# speedup vs baseline: 16.3280x; 16.3280x over previous
import functools

import jax
import jax.numpy as jnp
from jax.experimental import pallas as pl
from jax.experimental.pallas import tpu as pltpu

TDM = 100.0
LANES = 128


def _mm(a, w):
    return jax.lax.dot_general(
        a, w, (((1,), (1,)), ((), ())),
        preferred_element_type=jnp.float32,
        precision=jax.lax.Precision.HIGHEST)


def _body(tb_ref, z_ref, u_ref, nb_ref, tc_ref, wt_ref, alpha_ref, psi_ref,
          omgb_ref, omg_ref, whw_ref, whb_ref, we2nw_ref, we2nb_ref,
          wrecew_ref, wreceb_ref, wrecnw_ref, wrecnb_ref, wtrow_ref,
          wtb_ref, out_ref, z_s, g_s, *, K, R):
    b = pl.program_id(0)

    @pl.when(b == 0)
    def _init():
        z_s[...] = z_ref[...]
        omg = omg_ref[...]
        ob = omgb_ref[0]
        for r in range(R):
            zc = z_ref[r * LANES:(r + 1) * LANES, :]
            g_s[r:r + 1, :] = _mm(omg, zc) + ob

    tb = tb_ref[0]
    tc = tc_ref[b]
    gcur = g_s[...]
    t_all = jnp.clip(tc - tb, 0.0, TDM) / TDM
    psi_c = psi_ref[0]
    gt = gcur + wt_ref[0] * t_all
    x = jnp.clip(alpha_ref[0] * gt / psi_c, -75.0, 75.0)
    out_ref[0] = psi_c * jnp.log1p(jnp.exp(x))

    ue = u_ref[b]
    nbs = [nb_ref[b, k] for k in range(K)]
    idxs = [ue] + nbs

    zu = z_s[pl.ds(ue, 1), :]
    znb = jnp.concatenate(
        [z_s[pl.ds(i, 1), :] for i in nbs], axis=0)

    lane_iota = jax.lax.broadcasted_iota(jnp.int32, (1, LANES), 1)

    def getval(idx):
        r = idx // LANES
        l = idx % LANES
        row = tb_ref[0, pl.ds(r, 1), :]
        return jnp.sum(jnp.where(lane_iota == l, row, 0.0), axis=1,
                       keepdims=True)

    tb_u = getval(ue)
    tb_nb = jnp.concatenate([getval(i) for i in nbs], axis=0)

    wtrow = wtrow_ref[...]
    wtb = wtb_ref[...]
    wt_u = (jnp.clip(tc - tb_u, 0.0, TDM) / TDM) * wtrow + wtb
    wt_nb = (jnp.clip(tc - tb_nb, 0.0, TDM) / TDM) * wtrow + wtb

    z17 = jnp.concatenate([zu, znb], axis=0)
    wh = _mm(z17, whw_ref[...])
    rec_e = _mm(zu, wrecew_ref[...]) + wreceb_ref[...]
    msg = _mm(zu, we2nw_ref[...]) + we2nb_ref[...]
    rec_n = _mm(znb, wrecnw_ref[...]) + wrecnb_ref[...]

    whb = whb_ref[...]
    z_new_u = jnp.tanh(rec_e + wt_u + wh[0:1, :] + whb)
    z_new_nb = jnp.tanh(rec_n + msg + wt_nb + wh[1:, :] + whb)

    znew = jnp.concatenate([z_new_u, z_new_nb], axis=0)
    gv = _mm(omg_ref[...], znew) + omgb_ref[0]

    z_s[pl.ds(ue, 1), :] = z_new_u
    for k in range(K):
        z_s[pl.ds(nbs[k], 1), :] = z_new_nb[k:k + 1, :]

    for j, idx in enumerate(idxs):
        r = idx // LANES
        l = idx % LANES
        row = g_s[pl.ds(r, 1), :]
        g_s[pl.ds(r, 1), :] = jnp.where(lane_iota == l,
                                        gv[0:1, j:j + 1], row)


def kernel(u, time_delta, time_bar, time_cur, significance, magnitudo, z,
           neigh, w_t, alpha, psi, omega_w, omega_b, W_h_w, W_h_b, W_e2n_w,
           W_e2n_b, W_rec_e_w, W_rec_e_b, W_rec_n_w, W_rec_n_b, W_t_w,
           W_t_b):
    N, D = z.shape
    B = u.shape[0]
    K = neigh.shape[1]
    R = (N + LANES - 1) // LANES
    NP = R * LANES

    u32 = u.astype(jnp.int32)
    nb_all = neigh[u32].astype(jnp.int32)
    z_pad = jnp.pad(z, ((0, NP - N), (0, 0)))
    tb_pad = jnp.pad(time_bar, ((0, 0), (0, NP - N))).reshape(B, R, LANES)

    row = lambda v: v.reshape(1, -1)
    wtrow = W_t_w[:, 0].reshape(1, D)

    smem = pl.BlockSpec(memory_space=pltpu.SMEM)
    full = lambda a: pl.BlockSpec(a.shape, lambda b: (0,) * a.ndim)

    body = functools.partial(_body, K=K, R=R)
    out = pl.pallas_call(
        body,
        grid=(B,),
        in_specs=[
            pl.BlockSpec((1, R, LANES), lambda b: (b, 0, 0)),
            full(z_pad),
            smem,
            smem,
            smem,
            smem,
            smem,
            smem,
            smem,
            full(omega_w),
            full(W_h_w), full(row(W_h_b)),
            full(W_e2n_w), full(row(W_e2n_b)),
            full(W_rec_e_w), full(row(W_rec_e_b)),
            full(W_rec_n_w), full(row(W_rec_n_b)),
            full(wtrow), full(row(W_t_b)),
        ],
        out_specs=pl.BlockSpec((1, R, LANES), lambda b: (b, 0, 0)),
        out_shape=jax.ShapeDtypeStruct((B, R, LANES), jnp.float32),
        scratch_shapes=[
            pltpu.VMEM((NP, D), jnp.float32),
            pltpu.VMEM((R, LANES), jnp.float32),
        ],
        compiler_params=pltpu.CompilerParams(
            dimension_semantics=("arbitrary",)),
    )(tb_pad, z_pad, u32, nb_all, time_cur, w_t, alpha, psi, omega_b,
      omega_w, W_h_w, row(W_h_b), W_e2n_w, row(W_e2n_b), W_rec_e_w,
      row(W_rec_e_b), W_rec_n_w, row(W_rec_n_b), wtrow, row(W_t_b))

    return out.reshape(B, NP)[:, :N]


# speedup vs baseline: 16.5700x; 1.0148x over previous
import functools

import jax
import jax.numpy as jnp
from jax.experimental import pallas as pl
from jax.experimental.pallas import tpu as pltpu

TDM = 100.0
LANES = 128


def _mm(a, w):
    return jax.lax.dot_general(
        a, w, (((1,), (1,)), ((), ())),
        preferred_element_type=jnp.float32,
        precision=jax.lax.Precision.HIGHEST)


def _body(tb_ref, z_ref, u_ref, nb_ref, tc_ref, wt_ref, alpha_ref, psi_ref,
          omgb_ref, omg_ref, wu2_ref, wnb_ref, bu_ref, bnb_ref, wtrow_ref,
          out_ref, z_s, g_s, *, K, R):
    b = pl.program_id(0)

    @pl.when(b == 0)
    def _init():
        z_s[...] = z_ref[...]
        omg = omg_ref[...]
        ob = omgb_ref[0]
        for r in range(R):
            zc = z_ref[r * LANES:(r + 1) * LANES, :]
            g_s[r:r + 1, :] = _mm(omg, zc) + ob

    tc = tc_ref[b]
    gcur = g_s[...]
    t_all = jnp.clip(tc - tb_ref[0], 0.0, TDM) / TDM
    psi_c = psi_ref[0]
    gt = gcur + wt_ref[0] * t_all
    x = jnp.clip(alpha_ref[0] * gt / psi_c, -75.0, 75.0)
    out_ref[0] = psi_c * jnp.log1p(jnp.exp(x))

    ue = u_ref[b]
    nbs = [nb_ref[b, k] for k in range(K)]
    idxs = [ue] + nbs
    rls = [(i // LANES, i % LANES) for i in idxs]

    zu = z_s[pl.ds(ue, 1), :]
    znb = jnp.concatenate(
        [z_s[pl.ds(i, 1), :] for i in nbs], axis=0)

    lane_iota = jax.lax.broadcasted_iota(jnp.int32, (1, LANES), 1)
    masked = jnp.concatenate(
        [jnp.where(lane_iota == l, tb_ref[0, pl.ds(r, 1), :], 0.0)
         for r, l in rls], axis=0)
    ones = jnp.ones((1, LANES), jnp.float32)
    tbv = _mm(ones, masked)
    t = jnp.clip(tc - tbv, 0.0, TDM) / TDM
    t_u = t[0:1, 0:1]
    t_nb = jnp.concatenate(
        [t[0:1, j:j + 1] for j in range(1, K + 1)], axis=0)

    wtrow = wtrow_ref[...]
    mm_u = _mm(zu, wu2_ref[...])
    D = LANES
    a_u = mm_u[:, 0:D]
    a_msg = mm_u[:, D:2 * D]

    z_new_u = jnp.tanh(a_u + t_u * wtrow + bu_ref[...])
    z_new_nb = jnp.tanh(_mm(znb, wnb_ref[...]) + a_msg
                        + t_nb * wtrow + bnb_ref[...])

    znew = jnp.concatenate([z_new_u, z_new_nb], axis=0)
    gv = _mm(omg_ref[...], znew) + omgb_ref[0]

    z_s[pl.ds(ue, 1), :] = z_new_u
    for k in range(K):
        z_s[pl.ds(nbs[k], 1), :] = z_new_nb[k:k + 1, :]

    for j, (r, l) in enumerate(rls):
        row = g_s[pl.ds(r, 1), :]
        g_s[pl.ds(r, 1), :] = jnp.where(lane_iota == l,
                                        gv[0:1, j:j + 1], row)


def kernel(u, time_delta, time_bar, time_cur, significance, magnitudo, z,
           neigh, w_t, alpha, psi, omega_w, omega_b, W_h_w, W_h_b, W_e2n_w,
           W_e2n_b, W_rec_e_w, W_rec_e_b, W_rec_n_w, W_rec_n_b, W_t_w,
           W_t_b):
    N, D = z.shape
    B = u.shape[0]
    K = neigh.shape[1]
    R = (N + LANES - 1) // LANES
    NP = R * LANES

    u32 = u.astype(jnp.int32)
    nb_all = neigh[u32].astype(jnp.int32)
    z_pad = jnp.pad(z, ((0, NP - N), (0, 0)))
    tb_pad = jnp.pad(time_bar, ((0, 0), (0, NP - N))).reshape(B, R, LANES)

    wu2 = jnp.concatenate([W_rec_e_w + W_h_w, W_e2n_w], axis=0)
    wnb = W_rec_n_w + W_h_w
    bu = (W_rec_e_b + W_h_b + W_t_b).reshape(1, D)
    bnb = (W_rec_n_b + W_h_b + W_t_b + W_e2n_b).reshape(1, D)
    wtrow = W_t_w[:, 0].reshape(1, D)

    smem = pl.BlockSpec(memory_space=pltpu.SMEM)
    full = lambda a: pl.BlockSpec(a.shape, lambda b: (0,) * a.ndim)

    body = functools.partial(_body, K=K, R=R)
    out = pl.pallas_call(
        body,
        grid=(B,),
        in_specs=[
            pl.BlockSpec((1, R, LANES), lambda b: (b, 0, 0)),
            full(z_pad),
            smem,
            smem,
            smem,
            smem,
            smem,
            smem,
            smem,
            full(omega_w),
            full(wu2), full(wnb), full(bu), full(bnb), full(wtrow),
        ],
        out_specs=pl.BlockSpec((1, R, LANES), lambda b: (b, 0, 0)),
        out_shape=jax.ShapeDtypeStruct((B, R, LANES), jnp.float32),
        scratch_shapes=[
            pltpu.VMEM((NP, D), jnp.float32),
            pltpu.VMEM((R, LANES), jnp.float32),
        ],
        compiler_params=pltpu.CompilerParams(
            dimension_semantics=("arbitrary",)),
    )(tb_pad, z_pad, u32, nb_all, time_cur, w_t, alpha, psi, omega_b,
      omega_w, wu2, wnb, bu, bnb, wtrow)

    return out.reshape(B, NP)[:, :N]


# speedup vs baseline: 18.1617x; 1.0961x over previous
import functools

import jax
import jax.numpy as jnp
from jax.experimental import pallas as pl
from jax.experimental.pallas import tpu as pltpu

TDM = 100.0
LANES = 128


def _mm(a, w):
    return jax.lax.dot_general(
        a, w, (((1,), (1,)), ((), ())),
        preferred_element_type=jnp.float32,
        precision=jax.lax.Precision.DEFAULT)


def _body(tb_ref, z_ref, u_ref, nb_ref, tc_ref, wt_ref, alpha_ref, psi_ref,
          omgb_ref, omg_ref, wu2_ref, wnb_ref, bu_ref, bnb_ref, wtrow_ref,
          out_ref, z_s, g_s, *, K, R):
    b = pl.program_id(0)

    @pl.when(b == 0)
    def _init():
        z_s[...] = z_ref[...]
        omg = omg_ref[...]
        ob = omgb_ref[0]
        for r in range(R):
            zc = z_ref[r * LANES:(r + 1) * LANES, :]
            g_s[r:r + 1, :] = _mm(omg, zc) + ob

    tc = tc_ref[b]
    gcur = g_s[...]
    t_all = jnp.clip(tc - tb_ref[0], 0.0, TDM) / TDM
    psi_c = psi_ref[0]
    gt = gcur + wt_ref[0] * t_all
    x = jnp.clip(alpha_ref[0] * gt / psi_c, -75.0, 75.0)
    out_ref[0] = psi_c * jnp.log1p(jnp.exp(x))

    ue = u_ref[b]
    nbs = [nb_ref[b, k] for k in range(K)]
    idxs = [ue] + nbs
    rls = [(i // LANES, i % LANES) for i in idxs]

    zu = z_s[pl.ds(ue, 1), :]
    znb = jnp.concatenate(
        [z_s[pl.ds(i, 1), :] for i in nbs], axis=0)

    lane_iota = jax.lax.broadcasted_iota(jnp.int32, (1, LANES), 1)
    masked = jnp.concatenate(
        [jnp.where(lane_iota == l, tb_ref[0, pl.ds(r, 1), :], 0.0)
         for r, l in rls], axis=0)
    ones = jnp.ones((1, LANES), jnp.float32)
    tbv = _mm(ones, masked)
    t = jnp.clip(tc - tbv, 0.0, TDM) / TDM
    t_u = t[0:1, 0:1]
    t_nb = jnp.concatenate(
        [t[0:1, j:j + 1] for j in range(1, K + 1)], axis=0)

    wtrow = wtrow_ref[...]
    mm_u = _mm(zu, wu2_ref[...])
    D = LANES
    a_u = mm_u[:, 0:D]
    a_msg = mm_u[:, D:2 * D]

    z_new_u = jnp.tanh(a_u + t_u * wtrow + bu_ref[...])
    z_new_nb = jnp.tanh(_mm(znb, wnb_ref[...]) + a_msg
                        + t_nb * wtrow + bnb_ref[...])

    znew = jnp.concatenate([z_new_u, z_new_nb], axis=0)
    gv = _mm(omg_ref[...], znew) + omgb_ref[0]

    z_s[pl.ds(ue, 1), :] = z_new_u
    for k in range(K):
        z_s[pl.ds(nbs[k], 1), :] = z_new_nb[k:k + 1, :]

    for j, (r, l) in enumerate(rls):
        row = g_s[pl.ds(r, 1), :]
        g_s[pl.ds(r, 1), :] = jnp.where(lane_iota == l,
                                        gv[0:1, j:j + 1], row)


def kernel(u, time_delta, time_bar, time_cur, significance, magnitudo, z,
           neigh, w_t, alpha, psi, omega_w, omega_b, W_h_w, W_h_b, W_e2n_w,
           W_e2n_b, W_rec_e_w, W_rec_e_b, W_rec_n_w, W_rec_n_b, W_t_w,
           W_t_b):
    N, D = z.shape
    B = u.shape[0]
    K = neigh.shape[1]
    R = (N + LANES - 1) // LANES
    NP = R * LANES

    u32 = u.astype(jnp.int32)
    nb_all = neigh[u32].astype(jnp.int32)
    z_pad = jnp.pad(z, ((0, NP - N), (0, 0)))
    tb_pad = jnp.pad(time_bar, ((0, 0), (0, NP - N))).reshape(B, R, LANES)

    wu2 = jnp.concatenate([W_rec_e_w + W_h_w, W_e2n_w], axis=0)
    wnb = W_rec_n_w + W_h_w
    bu = (W_rec_e_b + W_h_b + W_t_b).reshape(1, D)
    bnb = (W_rec_n_b + W_h_b + W_t_b + W_e2n_b).reshape(1, D)
    wtrow = W_t_w[:, 0].reshape(1, D)

    smem = pl.BlockSpec(memory_space=pltpu.SMEM)
    full = lambda a: pl.BlockSpec(a.shape, lambda b: (0,) * a.ndim)

    body = functools.partial(_body, K=K, R=R)
    out = pl.pallas_call(
        body,
        grid=(B,),
        in_specs=[
            pl.BlockSpec((1, R, LANES), lambda b: (b, 0, 0)),
            full(z_pad),
            smem,
            smem,
            smem,
            smem,
            smem,
            smem,
            smem,
            full(omega_w),
            full(wu2), full(wnb), full(bu), full(bnb), full(wtrow),
        ],
        out_specs=pl.BlockSpec((1, R, LANES), lambda b: (b, 0, 0)),
        out_shape=jax.ShapeDtypeStruct((B, R, LANES), jnp.float32),
        scratch_shapes=[
            pltpu.VMEM((NP, D), jnp.float32),
            pltpu.VMEM((R, LANES), jnp.float32),
        ],
        compiler_params=pltpu.CompilerParams(
            dimension_semantics=("arbitrary",)),
    )(tb_pad, z_pad, u32, nb_all, time_cur, w_t, alpha, psi, omega_b,
      omega_w, wu2, wnb, bu, bnb, wtrow)

    return out.reshape(B, NP)[:, :N]


# speedup vs baseline: 36.1113x; 1.9883x over previous
import functools

import jax
import jax.numpy as jnp
from jax.experimental import pallas as pl
from jax.experimental.pallas import tpu as pltpu

TDM = 100.0
LANES = 128


def _mm(a, w):
    return jax.lax.dot_general(
        a, w, (((1,), (1,)), ((), ())),
        preferred_element_type=jnp.float32,
        precision=jax.lax.Precision.DEFAULT)


def _body(tb_ref, z_ref, u_ref, nb_ref, tc_ref, wt_ref, alpha_ref, psi_ref,
          omgb_ref, omg_ref, wu2_ref, wnb_ref, bu_ref, bnb_ref, wtrow_ref,
          out_ref, z_s, g_s, *, K, R, G):
    pid = pl.program_id(0)

    @pl.when(pid == 0)
    def _init():
        z_s[...] = z_ref[...]
        omg = omg_ref[...]
        ob = omgb_ref[0]
        for r in range(R):
            zc = z_ref[r * LANES:(r + 1) * LANES, :]
            g_s[r:r + 1, :] = _mm(omg, zc) + ob

    for j in range(G):
        _event(tb_ref, u_ref, nb_ref, tc_ref, wt_ref, alpha_ref, psi_ref,
               omgb_ref, omg_ref, wu2_ref, wnb_ref, bu_ref, bnb_ref,
               wtrow_ref, out_ref, z_s, g_s, pid * G + j, j, K)


def _event(tb_ref, u_ref, nb_ref, tc_ref, wt_ref, alpha_ref, psi_ref,
           omgb_ref, omg_ref, wu2_ref, wnb_ref, bu_ref, bnb_ref, wtrow_ref,
           out_ref, z_s, g_s, b, j, K):
    tc = tc_ref[b]
    gcur = g_s[...]
    t_all = jnp.clip(tc - tb_ref[j], 0.0, TDM) / TDM
    psi_c = psi_ref[0]
    gt = gcur + wt_ref[0] * t_all
    x = jnp.clip(alpha_ref[0] * gt / psi_c, -75.0, 75.0)
    out_ref[j] = psi_c * jnp.log1p(jnp.exp(x))

    ue = u_ref[b]
    nbs = [nb_ref[b, k] for k in range(K)]
    idxs = [ue] + nbs
    rls = [(i // LANES, i % LANES) for i in idxs]

    zu = z_s[pl.ds(ue, 1), :]
    znb = jnp.concatenate(
        [z_s[pl.ds(i, 1), :] for i in nbs], axis=0)

    lane_iota = jax.lax.broadcasted_iota(jnp.int32, (1, LANES), 1)
    masked = jnp.concatenate(
        [jnp.where(lane_iota == l, tb_ref[j, pl.ds(r, 1), :], 0.0)
         for r, l in rls], axis=0)
    ones = jnp.ones((1, LANES), jnp.float32)
    tbv = _mm(ones, masked)
    t = jnp.clip(tc - tbv, 0.0, TDM) / TDM
    t_u = t[0:1, 0:1]
    t_nb = jnp.concatenate(
        [t[0:1, j:j + 1] for j in range(1, K + 1)], axis=0)

    wtrow = wtrow_ref[...]
    mm_u = _mm(zu, wu2_ref[...])
    D = LANES
    a_u = mm_u[:, 0:D]
    a_msg = mm_u[:, D:2 * D]

    z_new_u = jnp.tanh(a_u + t_u * wtrow + bu_ref[...])
    z_new_nb = jnp.tanh(_mm(znb, wnb_ref[...]) + a_msg
                        + t_nb * wtrow + bnb_ref[...])

    znew = jnp.concatenate([z_new_u, z_new_nb], axis=0)
    gv = _mm(omg_ref[...], znew) + omgb_ref[0]

    z_s[pl.ds(ue, 1), :] = z_new_u
    for k in range(K):
        z_s[pl.ds(nbs[k], 1), :] = z_new_nb[k:k + 1, :]

    for jj, (r, l) in enumerate(rls):
        row = g_s[pl.ds(r, 1), :]
        g_s[pl.ds(r, 1), :] = jnp.where(lane_iota == l,
                                        gv[0:1, jj:jj + 1], row)


def kernel(u, time_delta, time_bar, time_cur, significance, magnitudo, z,
           neigh, w_t, alpha, psi, omega_w, omega_b, W_h_w, W_h_b, W_e2n_w,
           W_e2n_b, W_rec_e_w, W_rec_e_b, W_rec_n_w, W_rec_n_b, W_t_w,
           W_t_b):
    N, D = z.shape
    B = u.shape[0]
    K = neigh.shape[1]
    R = (N + LANES - 1) // LANES
    NP = R * LANES

    u32 = u.astype(jnp.int32)
    nb_all = neigh[u32].astype(jnp.int32)
    z_pad = jnp.pad(z, ((0, NP - N), (0, 0)))
    tb_pad = jnp.pad(time_bar, ((0, 0), (0, NP - N))).reshape(B, R, LANES)

    wu2 = jnp.concatenate([W_rec_e_w + W_h_w, W_e2n_w], axis=0)
    wnb = W_rec_n_w + W_h_w
    bu = (W_rec_e_b + W_h_b + W_t_b).reshape(1, D)
    bnb = (W_rec_n_b + W_h_b + W_t_b + W_e2n_b).reshape(1, D)
    wtrow = W_t_w[:, 0].reshape(1, D)

    smem = pl.BlockSpec(memory_space=pltpu.SMEM)
    full = lambda a: pl.BlockSpec(a.shape, lambda b: (0,) * a.ndim)

    G = 8 if B % 8 == 0 else 1
    body = functools.partial(_body, K=K, R=R, G=G)
    out = pl.pallas_call(
        body,
        grid=(B // G,),
        in_specs=[
            pl.BlockSpec((G, R, LANES), lambda b: (b, 0, 0)),
            full(z_pad),
            smem,
            smem,
            smem,
            smem,
            smem,
            smem,
            smem,
            full(omega_w),
            full(wu2), full(wnb), full(bu), full(bnb), full(wtrow),
        ],
        out_specs=pl.BlockSpec((G, R, LANES), lambda b: (b, 0, 0)),
        out_shape=jax.ShapeDtypeStruct((B, R, LANES), jnp.float32),
        scratch_shapes=[
            pltpu.VMEM((NP, D), jnp.float32),
            pltpu.VMEM((R, LANES), jnp.float32),
        ],
        compiler_params=pltpu.CompilerParams(
            dimension_semantics=("arbitrary",)),
    )(tb_pad, z_pad, u32, nb_all, time_cur, w_t, alpha, psi, omega_b,
      omega_w, wu2, wnb, bu, bnb, wtrow)

    return out.reshape(B, NP)[:, :N]


# speedup vs baseline: 37.6689x; 1.0431x over previous
import functools

import jax
import jax.numpy as jnp
from jax.experimental import pallas as pl
from jax.experimental.pallas import tpu as pltpu

TDM = 100.0
LANES = 128


def _mm(a, w):
    return jax.lax.dot_general(
        a, w, (((1,), (1,)), ((), ())),
        preferred_element_type=jnp.float32,
        precision=jax.lax.Precision.DEFAULT)


def _body(tb_ref, z_ref, u_ref, nb_ref, tc_ref, wt_ref, alpha_ref, psi_ref,
          omgb_ref, omg_ref, wu2_ref, wnb_ref, bu_ref, bnb_ref, wtrow_ref,
          out_ref, z_s, g_s, *, K, R, G):
    pid = pl.program_id(0)

    @pl.when(pid == 0)
    def _init():
        z_s[...] = z_ref[...]
        omg = omg_ref[...]
        ob = omgb_ref[0]
        for r in range(R):
            zc = z_ref[r * LANES:(r + 1) * LANES, :]
            g_s[r:r + 1, :] = _mm(omg, zc) + ob

    for j in range(G):
        _event(tb_ref, u_ref, nb_ref, tc_ref, wt_ref, alpha_ref, psi_ref,
               omgb_ref, omg_ref, wu2_ref, wnb_ref, bu_ref, bnb_ref,
               wtrow_ref, out_ref, z_s, g_s, pid * G + j, j, K)


def _event(tb_ref, u_ref, nb_ref, tc_ref, wt_ref, alpha_ref, psi_ref,
           omgb_ref, omg_ref, wu2_ref, wnb_ref, bu_ref, bnb_ref, wtrow_ref,
           out_ref, z_s, g_s, b, j, K):
    tc = tc_ref[b]
    gcur = g_s[...]
    t_all = jnp.clip(tc - tb_ref[j], 0.0, TDM) / TDM
    psi_c = psi_ref[0]
    gt = gcur + wt_ref[0] * t_all
    x = jnp.clip(alpha_ref[0] * gt / psi_c, -75.0, 75.0)
    out_ref[j] = psi_c * jnp.log1p(jnp.exp(x))

    ue = u_ref[b]
    nbs = [nb_ref[b, k] for k in range(K)]
    idxs = [ue] + nbs
    rls = [(i // LANES, i % LANES) for i in idxs]

    zu = z_s[pl.ds(ue, 1), :]
    znb = jnp.concatenate(
        [z_s[pl.ds(i, 1), :] for i in nbs], axis=0)

    lane_iota = jax.lax.broadcasted_iota(jnp.int32, (1, LANES), 1)
    masked = jnp.concatenate(
        [jnp.where(lane_iota == l, tb_ref[j, pl.ds(r, 1), :], 0.0)
         for r, l in rls], axis=0)
    ones = jnp.ones((1, LANES), jnp.float32)
    tbv = _mm(ones, masked)
    t = jnp.clip(tc - tbv, 0.0, TDM) / TDM
    t_u = t[0:1, 0:1]
    t_nb = jnp.concatenate(
        [t[0:1, j:j + 1] for j in range(1, K + 1)], axis=0)

    wtrow = wtrow_ref[...]
    mm_u = _mm(zu, wu2_ref[...])
    D = LANES
    a_u = mm_u[:, 0:D]
    a_msg = mm_u[:, D:2 * D]

    z_new_u = jnp.tanh(a_u + t_u * wtrow + bu_ref[...])
    z_new_nb = jnp.tanh(_mm(znb, wnb_ref[...]) + a_msg
                        + t_nb * wtrow + bnb_ref[...])

    znew = jnp.concatenate([z_new_u, z_new_nb], axis=0)
    gv = _mm(omg_ref[...], znew) + omgb_ref[0]

    z_s[pl.ds(ue, 1), :] = z_new_u
    for k in range(K):
        z_s[pl.ds(nbs[k], 1), :] = z_new_nb[k:k + 1, :]

    for jj, (r, l) in enumerate(rls):
        row = g_s[pl.ds(r, 1), :]
        g_s[pl.ds(r, 1), :] = jnp.where(lane_iota == l,
                                        gv[0:1, jj:jj + 1], row)


def kernel(u, time_delta, time_bar, time_cur, significance, magnitudo, z,
           neigh, w_t, alpha, psi, omega_w, omega_b, W_h_w, W_h_b, W_e2n_w,
           W_e2n_b, W_rec_e_w, W_rec_e_b, W_rec_n_w, W_rec_n_b, W_t_w,
           W_t_b):
    N, D = z.shape
    B = u.shape[0]
    K = neigh.shape[1]
    R = (N + LANES - 1) // LANES
    NP = R * LANES

    u32 = u.astype(jnp.int32)
    nb_all = neigh[u32].astype(jnp.int32)
    z_pad = jnp.pad(z, ((0, NP - N), (0, 0)))
    tb_pad = jnp.pad(time_bar, ((0, 0), (0, NP - N))).reshape(B, R, LANES)

    wu2 = jnp.concatenate([W_rec_e_w + W_h_w, W_e2n_w], axis=0)
    wnb = W_rec_n_w + W_h_w
    bu = (W_rec_e_b + W_h_b + W_t_b).reshape(1, D)
    bnb = (W_rec_n_b + W_h_b + W_t_b + W_e2n_b).reshape(1, D)
    wtrow = W_t_w[:, 0].reshape(1, D)

    smem = pl.BlockSpec(memory_space=pltpu.SMEM)
    full = lambda a: pl.BlockSpec(a.shape, lambda b: (0,) * a.ndim)

    G = 16 if B % 16 == 0 else (8 if B % 8 == 0 else 1)
    body = functools.partial(_body, K=K, R=R, G=G)
    out = pl.pallas_call(
        body,
        grid=(B // G,),
        in_specs=[
            pl.BlockSpec((G, R, LANES), lambda b: (b, 0, 0)),
            full(z_pad),
            smem,
            smem,
            smem,
            smem,
            smem,
            smem,
            smem,
            full(omega_w),
            full(wu2), full(wnb), full(bu), full(bnb), full(wtrow),
        ],
        out_specs=pl.BlockSpec((G, R, LANES), lambda b: (b, 0, 0)),
        out_shape=jax.ShapeDtypeStruct((B, R, LANES), jnp.float32),
        scratch_shapes=[
            pltpu.VMEM((NP, D), jnp.float32),
            pltpu.VMEM((R, LANES), jnp.float32),
        ],
        compiler_params=pltpu.CompilerParams(
            dimension_semantics=("arbitrary",)),
    )(tb_pad, z_pad, u32, nb_all, time_cur, w_t, alpha, psi, omega_b,
      omega_w, wu2, wnb, bu, bnb, wtrow)

    return out.reshape(B, NP)[:, :N]


# speedup vs baseline: 37.9808x; 1.0083x over previous
import functools

import jax
import jax.numpy as jnp
from jax.experimental import pallas as pl
from jax.experimental.pallas import tpu as pltpu

TDM = 100.0
LANES = 128


def _mm(a, w):
    return jax.lax.dot_general(
        a, w, (((1,), (1,)), ((), ())),
        preferred_element_type=jnp.float32,
        precision=jax.lax.Precision.DEFAULT)


def _body(tb_ref, z_ref, u_ref, nb_ref, tc_ref, wt_ref, alpha_ref, psi_ref,
          omgb_ref, omg_ref, wu2_ref, wnb_ref, bu_ref, bnb_ref, wtrow_ref,
          out_ref, z_s, g_s, *, K, R, G):
    pid = pl.program_id(0)

    @pl.when(pid == 0)
    def _init():
        z_s[...] = z_ref[...]
        omg = omg_ref[...]
        ob = omgb_ref[0]
        for r in range(R):
            zc = z_ref[r * LANES:(r + 1) * LANES, :]
            g_s[r:r + 1, :] = _mm(omg, zc) + ob

    for j in range(G):
        _event(tb_ref, u_ref, nb_ref, tc_ref, wt_ref, alpha_ref, psi_ref,
               omgb_ref, omg_ref, wu2_ref, wnb_ref, bu_ref, bnb_ref,
               wtrow_ref, out_ref, z_s, g_s, pid * G + j, j, K)


def _event(tb_ref, u_ref, nb_ref, tc_ref, wt_ref, alpha_ref, psi_ref,
           omgb_ref, omg_ref, wu2_ref, wnb_ref, bu_ref, bnb_ref, wtrow_ref,
           out_ref, z_s, g_s, b, j, K):
    tc = tc_ref[b]
    gcur = g_s[...]
    t_all = jnp.clip(tc - tb_ref[j], 0.0, TDM) / TDM
    psi_c = psi_ref[0]
    a_over_p = alpha_ref[0] / psi_c
    gt = gcur + wt_ref[0] * t_all
    x = jnp.clip(a_over_p * gt, -75.0, 75.0)
    out_ref[j] = psi_c * jnp.log1p(jnp.exp(x))

    ue = u_ref[b]
    nbs = [nb_ref[b, k] for k in range(K)]
    idxs = [ue] + nbs
    rls = [(i // LANES, i % LANES) for i in idxs]

    zu = z_s[pl.ds(ue, 1), :]
    znb = jnp.concatenate(
        [z_s[pl.ds(i, 1), :] for i in nbs], axis=0)

    lane_iota = jax.lax.broadcasted_iota(jnp.int32, (1, LANES), 1)
    masked = jnp.concatenate(
        [jnp.where(lane_iota == l, tb_ref[j, pl.ds(r, 1), :], 0.0)
         for r, l in rls], axis=0)
    ones = jnp.ones((1, LANES), jnp.float32)
    tbv = _mm(ones, masked)
    t = jnp.clip(tc - tbv, 0.0, TDM) / TDM
    t_u = t[0:1, 0:1]
    t_nb = jnp.concatenate(
        [t[0:1, j:j + 1] for j in range(1, K + 1)], axis=0)

    wtrow = wtrow_ref[...]
    mm_u = _mm(zu, wu2_ref[...])
    D = LANES
    a_u = mm_u[:, 0:D]
    a_msg = mm_u[:, D:2 * D]

    z_new_u = jnp.tanh(a_u + t_u * wtrow + bu_ref[...])
    z_new_nb = jnp.tanh(_mm(znb, wnb_ref[...]) + a_msg
                        + t_nb * wtrow + bnb_ref[...])

    znew = jnp.concatenate([z_new_u, z_new_nb], axis=0)
    gv = _mm(omg_ref[...], znew) + omgb_ref[0]

    z_s[pl.ds(ue, 1), :] = z_new_u
    for k in range(K):
        z_s[pl.ds(nbs[k], 1), :] = z_new_nb[k:k + 1, :]

    for jj, (r, l) in enumerate(rls):
        row = g_s[pl.ds(r, 1), :]
        g_s[pl.ds(r, 1), :] = jnp.where(lane_iota == l,
                                        gv[0:1, jj:jj + 1], row)


def kernel(u, time_delta, time_bar, time_cur, significance, magnitudo, z,
           neigh, w_t, alpha, psi, omega_w, omega_b, W_h_w, W_h_b, W_e2n_w,
           W_e2n_b, W_rec_e_w, W_rec_e_b, W_rec_n_w, W_rec_n_b, W_t_w,
           W_t_b):
    N, D = z.shape
    B = u.shape[0]
    K = neigh.shape[1]
    R = (N + LANES - 1) // LANES
    NP = R * LANES

    u32 = u.astype(jnp.int32)
    nb_all = neigh[u32].astype(jnp.int32)
    z_pad = jnp.pad(z, ((0, NP - N), (0, 0)))
    tb_pad = jnp.pad(time_bar, ((0, 0), (0, NP - N))).reshape(B, R, LANES)

    wu2 = jnp.concatenate([W_rec_e_w + W_h_w, W_e2n_w], axis=0)
    wnb = W_rec_n_w + W_h_w
    bu = (W_rec_e_b + W_h_b + W_t_b).reshape(1, D)
    bnb = (W_rec_n_b + W_h_b + W_t_b + W_e2n_b).reshape(1, D)
    wtrow = W_t_w[:, 0].reshape(1, D)

    smem = pl.BlockSpec(memory_space=pltpu.SMEM)
    full = lambda a: pl.BlockSpec(a.shape, lambda b: (0,) * a.ndim)

    G = 16 if B % 16 == 0 else (8 if B % 8 == 0 else 1)
    body = functools.partial(_body, K=K, R=R, G=G)
    out = pl.pallas_call(
        body,
        grid=(B // G,),
        in_specs=[
            pl.BlockSpec((G, R, LANES), lambda b: (b, 0, 0)),
            full(z_pad),
            smem,
            smem,
            smem,
            smem,
            smem,
            smem,
            smem,
            full(omega_w),
            full(wu2), full(wnb), full(bu), full(bnb), full(wtrow),
        ],
        out_specs=pl.BlockSpec((G, R, LANES), lambda b: (b, 0, 0)),
        out_shape=jax.ShapeDtypeStruct((B, R, LANES), jnp.float32),
        scratch_shapes=[
            pltpu.VMEM((NP, D), jnp.float32),
            pltpu.VMEM((R, LANES), jnp.float32),
        ],
        compiler_params=pltpu.CompilerParams(
            dimension_semantics=("arbitrary",)),
    )(tb_pad, z_pad, u32, nb_all, time_cur, w_t, alpha, psi, omega_b,
      omega_w, wu2, wnb, bu, bnb, wtrow)

    return out.reshape(B, NP)[:, :N]


# speedup vs baseline: 38.8149x; 1.0220x over previous
import functools

import jax
import jax.numpy as jnp
from jax.experimental import pallas as pl
from jax.experimental.pallas import tpu as pltpu

TDM = 100.0
LANES = 128


def _mm(a, w):
    return jax.lax.dot_general(
        a, w, (((1,), (1,)), ((), ())),
        preferred_element_type=jnp.float32,
        precision=jax.lax.Precision.DEFAULT)


def _body(tb_ref, z_ref, u_ref, nb_ref, tc_ref, wt_ref, alpha_ref, psi_ref,
          omgb_ref, omg_ref, wu2_ref, wnb_ref, bu_ref, bnb_ref, wtrow_ref,
          out_ref, z_s, g_s, *, K, R, G):
    pid = pl.program_id(0)

    @pl.when(pid == 0)
    def _init():
        z_s[...] = z_ref[...]
        omg = omg_ref[...]
        ob = omgb_ref[0]
        for r in range(R):
            zc = z_ref[r * LANES:(r + 1) * LANES, :]
            g_s[r:r + 1, :] = _mm(omg, zc) + ob

    for j in range(G):
        _event(tb_ref, u_ref, nb_ref, tc_ref, wt_ref, alpha_ref, psi_ref,
               omgb_ref, omg_ref, wu2_ref, wnb_ref, bu_ref, bnb_ref,
               wtrow_ref, out_ref, z_s, g_s, pid * G + j, j, K)


def _event(tb_ref, u_ref, nb_ref, tc_ref, wt_ref, alpha_ref, psi_ref,
           omgb_ref, omg_ref, wu2_ref, wnb_ref, bu_ref, bnb_ref, wtrow_ref,
           out_ref, z_s, g_s, b, j, K):
    tc = tc_ref[b]
    gcur = g_s[...]
    t_all = jnp.clip(tc - tb_ref[j], 0.0, TDM) / TDM
    psi_c = psi_ref[0]
    a_over_p = alpha_ref[0] / psi_c
    gt = gcur + wt_ref[0] * t_all
    x = jnp.clip(a_over_p * gt, -75.0, 75.0)
    out_ref[j] = psi_c * jnp.log1p(jnp.exp(x))

    ue = u_ref[b]
    nbs = [nb_ref[b, k] for k in range(K)]
    idxs = [ue] + nbs
    rls = [(i // LANES, i % LANES) for i in idxs]

    zu = z_s[pl.ds(ue, 1), :]
    znb = jnp.concatenate(
        [z_s[pl.ds(i, 1), :] for i in nbs], axis=0)

    lane_iota = jax.lax.broadcasted_iota(jnp.int32, (1, LANES), 1)
    masked = jnp.concatenate(
        [jnp.where(lane_iota == l, tb_ref[j, pl.ds(r, 1), :], 0.0)
         for r, l in rls], axis=0)
    ones = jnp.ones((1, LANES), jnp.float32)
    tbv = _mm(ones, masked)
    t = jnp.clip(tc - tbv, 0.0, TDM) / TDM
    t_u = t[0:1, 0:1]
    t_nb = jnp.concatenate(
        [t[0:1, j:j + 1] for j in range(1, K + 1)], axis=0)

    wtrow = wtrow_ref[...]
    mm_u = _mm(zu, wu2_ref[...])
    D = LANES
    a_u = mm_u[:, 0:D]
    a_msg = mm_u[:, D:2 * D]

    z_new_u = jnp.tanh(a_u + t_u * wtrow + bu_ref[...])
    z_new_nb = jnp.tanh(_mm(znb, wnb_ref[...]) + a_msg
                        + t_nb * wtrow + bnb_ref[...])

    znew = jnp.concatenate([z_new_u, z_new_nb], axis=0)
    gv = _mm(omg_ref[...], znew) + omgb_ref[0]

    z_s[pl.ds(ue, 1), :] = z_new_u
    for k in range(K):
        z_s[pl.ds(nbs[k], 1), :] = z_new_nb[k:k + 1, :]

    for jj, (r, l) in enumerate(rls):
        row = g_s[pl.ds(r, 1), :]
        g_s[pl.ds(r, 1), :] = jnp.where(lane_iota == l,
                                        gv[0:1, jj:jj + 1], row)


def kernel(u, time_delta, time_bar, time_cur, significance, magnitudo, z,
           neigh, w_t, alpha, psi, omega_w, omega_b, W_h_w, W_h_b, W_e2n_w,
           W_e2n_b, W_rec_e_w, W_rec_e_b, W_rec_n_w, W_rec_n_b, W_t_w,
           W_t_b):
    N, D = z.shape
    B = u.shape[0]
    K = neigh.shape[1]
    R = (N + LANES - 1) // LANES
    NP = R * LANES

    u32 = u.astype(jnp.int32)
    nb_all = neigh[u32].astype(jnp.int32)
    z_pad = jnp.pad(z, ((0, NP - N), (0, 0)))
    tb_pad = jnp.pad(time_bar, ((0, 0), (0, NP - N))).reshape(B, R, LANES)

    wu2 = jnp.concatenate([W_rec_e_w + W_h_w, W_e2n_w], axis=0)
    wnb = W_rec_n_w + W_h_w
    bu = (W_rec_e_b + W_h_b + W_t_b).reshape(1, D)
    bnb = (W_rec_n_b + W_h_b + W_t_b + W_e2n_b).reshape(1, D)
    wtrow = W_t_w[:, 0].reshape(1, D)

    smem = pl.BlockSpec(memory_space=pltpu.SMEM)
    full = lambda a: pl.BlockSpec(a.shape, lambda b: (0,) * a.ndim)

    G = 32 if B % 32 == 0 else (8 if B % 8 == 0 else 1)
    body = functools.partial(_body, K=K, R=R, G=G)
    out = pl.pallas_call(
        body,
        grid=(B // G,),
        in_specs=[
            pl.BlockSpec((G, R, LANES), lambda b: (b, 0, 0)),
            full(z_pad),
            smem,
            smem,
            smem,
            smem,
            smem,
            smem,
            smem,
            full(omega_w),
            full(wu2), full(wnb), full(bu), full(bnb), full(wtrow),
        ],
        out_specs=pl.BlockSpec((G, R, LANES), lambda b: (b, 0, 0)),
        out_shape=jax.ShapeDtypeStruct((B, R, LANES), jnp.float32),
        scratch_shapes=[
            pltpu.VMEM((NP, D), jnp.float32),
            pltpu.VMEM((R, LANES), jnp.float32),
        ],
        compiler_params=pltpu.CompilerParams(
            dimension_semantics=("arbitrary",)),
    )(tb_pad, z_pad, u32, nb_all, time_cur, w_t, alpha, psi, omega_b,
      omega_w, wu2, wnb, bu, bnb, wtrow)

    return out.reshape(B, NP)[:, :N]
